# Initial kernel scaffold; baseline (speedup 1.0000x reference)
import jax
import jax.numpy as jnp
from jax.experimental import pallas as pl


def kernel(feat, gamma, beta, W_u, W_v, b_v, W_e, segment_ids, last_nodes):
    raise NotImplementedError("write your pallas kernel here")


# speedup vs baseline: 8.1065x; 8.1065x over previous
import jax
import jax.numpy as jnp
from jax.experimental import pallas as pl
from jax.experimental.pallas import tpu as pltpu

_N = 32768
_D = 128
_H = 128
_B = 16
_EPS = 1e-5
_C = 2048
_NB = _N // _C

_T00 = (((0,), (0,)), ((), ()))


def _attn_readout_kernel(seg_ref, ln_ref, feat_ref, gamma_ref, beta_ref,
                         wu_ref, wv_ref, bv_ref, we_ref,
                         out_ref,
                         sum_s, sq_s, fl_s, m_s, s_s, acc_s, sc_s, sh_s, fv_s):
    p = pl.program_id(0)
    i = pl.program_id(1)

    @pl.when((p == 0) & (i == 0))
    def _init():
        sum_s[...] = jnp.zeros_like(sum_s)
        sq_s[...] = jnp.zeros_like(sq_s)
        fl_s[...] = jnp.zeros_like(fl_s)
        m_s[...] = jnp.full_like(m_s, -jnp.inf)
        s_s[...] = jnp.zeros_like(s_s)
        acc_s[...] = jnp.zeros_like(acc_s)

    feat = feat_ref[...]

    @pl.when(p == 0)
    def _phase0():
        sum_s[...] += jnp.sum(feat, axis=0, keepdims=True)
        sq_s[...] += jnp.sum(feat * feat, axis=0, keepdims=True)
        rows = i * _C + jax.lax.broadcasted_iota(jnp.int32, (_C, 1), 0)
        rs = (rows == ln_ref[...]).astype(jnp.float32)
        fl_s[...] += jax.lax.dot_general(rs, feat, _T00)

    @pl.when((p == 1) & (i == 0))
    def _mid():
        mean = sum_s[...] / _N
        var = jnp.maximum(sq_s[...] / _N - mean * mean, 0.0)
        scale = gamma_ref[...] * jax.lax.rsqrt(var + _EPS)
        shift = beta_ref[...] - mean * scale
        sc_s[...] = scale
        sh_s[...] = shift
        fb_last = fl_s[...] * scale + shift
        fv_s[...] = jax.lax.dot(fb_last, wv_ref[...]) + bv_ref[...]

    @pl.when(p == 1)
    def _phase1():
        fb = feat * sc_s[...] + sh_s[...]
        u = jax.lax.dot(fb, wu_ref[...])

        seg_row = seg_ref[...][0]
        maskT = (jax.lax.broadcasted_iota(jnp.int32, (_B, 1), 0)
                 == seg_row).astype(jnp.float32)
        bvals = jax.lax.broadcasted_iota(
            jnp.int32, (_B, 1), 0).astype(jnp.float32)
        seg_col = jax.lax.dot_general(maskT, bvals, _T00)
        segm = (jax.lax.broadcasted_iota(jnp.int32, (1, _B), 1)
                == seg_col.astype(jnp.int32))
        maskf = segm.astype(jnp.float32)

        vb = jax.lax.dot(maskf, fv_s[...])
        sg = jax.nn.sigmoid(u + vb)
        e = jnp.sum(sg * we_ref[...], axis=1, keepdims=True)

        neg = jnp.float32(-jnp.inf)
        bm = jnp.max(jnp.where(segm, e, neg), axis=0, keepdims=True)
        m_old = m_s[...]
        m_new = jnp.maximum(m_old, bm)
        resc = jnp.where(m_old >= m_new, 1.0, jnp.exp(m_old - m_new))
        m_row = jnp.sum(maskf * m_new, axis=1, keepdims=True)
        ex = jnp.exp(e - m_row)
        w = maskf * ex
        s_s[...] = s_s[...] * resc + jnp.sum(w, axis=0, keepdims=True)
        eyeB = (jax.lax.broadcasted_iota(jnp.int32, (_B, _B), 0)
                == jax.lax.broadcasted_iota(jnp.int32, (_B, _B), 1))
        diag_resc = eyeB.astype(jnp.float32) * resc
        acc_s[...] = (jax.lax.dot(diag_resc, acc_s[...])
                      + jax.lax.dot_general(w, fb, _T00))
        m_s[...] = m_new

        @pl.when(i == _NB - 1)
        def _fin():
            sden = s_s[...]
            inv = jnp.where(sden > 0.0, 1.0 / sden, 0.0)
            out_ref[...] = jax.lax.dot(eyeB.astype(jnp.float32) * inv,
                                       acc_s[...])


def kernel(feat, gamma, beta, W_u, W_v, b_v, W_e, segment_ids, last_nodes):
    seg3 = segment_ids.astype(jnp.int32).reshape(_NB, 1, _C)
    ln = last_nodes.astype(jnp.int32).reshape(1, _B)
    g = gamma.reshape(1, _D).astype(jnp.float32)
    bt = beta.reshape(1, _D).astype(jnp.float32)
    bv = b_v.reshape(1, _H).astype(jnp.float32)
    we = W_e.reshape(1, _H).astype(jnp.float32)

    const = lambda p, i: (0, 0)
    out = pl.pallas_call(
        _attn_readout_kernel,
        grid=(2, _NB),
        in_specs=[
            pl.BlockSpec((1, 1, _C), lambda p, i: (i, 0, 0)),
            pl.BlockSpec((1, _B), const),
            pl.BlockSpec((_C, _D), lambda p, i: (i, 0)),
            pl.BlockSpec((1, _D), const),
            pl.BlockSpec((1, _D), const),
            pl.BlockSpec((_D, _H), const),
            pl.BlockSpec((_D, _H), const),
            pl.BlockSpec((1, _H), const),
            pl.BlockSpec((1, _H), const),
        ],
        out_specs=pl.BlockSpec((_B, _D), const),
        out_shape=jax.ShapeDtypeStruct((_B, _D), jnp.float32),
        scratch_shapes=[
            pltpu.VMEM((1, _D), jnp.float32),
            pltpu.VMEM((1, _D), jnp.float32),
            pltpu.VMEM((_B, _D), jnp.float32),
            pltpu.VMEM((1, _B), jnp.float32),
            pltpu.VMEM((1, _B), jnp.float32),
            pltpu.VMEM((_B, _D), jnp.float32),
            pltpu.VMEM((1, _D), jnp.float32),
            pltpu.VMEM((1, _D), jnp.float32),
            pltpu.VMEM((_B, _H), jnp.float32),
        ],
    )(seg3, ln, feat.astype(jnp.float32), g, bt,
      W_u.astype(jnp.float32), W_v.astype(jnp.float32), bv, we)
    return out


# speedup vs baseline: 8.2475x; 1.0174x over previous
import jax
import jax.numpy as jnp
from jax.experimental import pallas as pl
from jax.experimental.pallas import tpu as pltpu

_N = 32768
_D = 128
_H = 128
_B = 16
_EPS = 1e-5
_C = 2048
_NB = _N // _C

_T00 = (((0,), (0,)), ((), ()))


def _attn_readout_kernel(seg_ref, ln_ref, feat_ref, gamma_ref, beta_ref,
                         wu_ref, wv_ref, bv_ref, we_ref,
                         out_ref,
                         sum_s, sq_s, fl_s, m_s, s_s, acc_s, sc_s, sh_s, fv_s,
                         feat_s):
    p = pl.program_id(0)
    i = pl.program_id(1)

    @pl.when((p == 0) & (i == 0))
    def _init():
        sum_s[...] = jnp.zeros_like(sum_s)
        sq_s[...] = jnp.zeros_like(sq_s)
        fl_s[...] = jnp.zeros_like(fl_s)
        m_s[...] = jnp.full_like(m_s, -jnp.inf)
        s_s[...] = jnp.zeros_like(s_s)
        acc_s[...] = jnp.zeros_like(acc_s)

    @pl.when(p == 0)
    def _phase0():
        feat = feat_ref[...]
        feat_s[pl.ds(i * _C, _C), :] = feat
        sum_s[...] += jnp.sum(feat, axis=0, keepdims=True)
        sq_s[...] += jnp.sum(feat * feat, axis=0, keepdims=True)
        rows = i * _C + jax.lax.broadcasted_iota(jnp.int32, (_C, 1), 0)
        rs = (rows == ln_ref[...]).astype(jnp.float32)
        fl_s[...] += jax.lax.dot_general(rs, feat, _T00)

    @pl.when((p == 1) & (i == 0))
    def _mid():
        mean = sum_s[...] / _N
        var = jnp.maximum(sq_s[...] / _N - mean * mean, 0.0)
        scale = gamma_ref[...] * jax.lax.rsqrt(var + _EPS)
        shift = beta_ref[...] - mean * scale
        sc_s[...] = scale
        sh_s[...] = shift
        fb_last = fl_s[...] * scale + shift
        fv_s[...] = jax.lax.dot(fb_last, wv_ref[...]) + bv_ref[...]

    @pl.when(p == 1)
    def _phase1():
        feat = feat_s[pl.ds(i * _C, _C), :]
        fb = feat * sc_s[...] + sh_s[...]
        u = jax.lax.dot(fb, wu_ref[...])

        seg_row = seg_ref[...][0]
        maskT = (jax.lax.broadcasted_iota(jnp.int32, (_B, 1), 0)
                 == seg_row).astype(jnp.float32)
        bvals = jax.lax.broadcasted_iota(
            jnp.int32, (_B, 1), 0).astype(jnp.float32)
        seg_col = jax.lax.dot_general(maskT, bvals, _T00)
        segm = (jax.lax.broadcasted_iota(jnp.int32, (1, _B), 1)
                == seg_col.astype(jnp.int32))
        maskf = segm.astype(jnp.float32)

        vb = jax.lax.dot(maskf, fv_s[...])
        sg = jax.nn.sigmoid(u + vb)
        e = jnp.sum(sg * we_ref[...], axis=1, keepdims=True)

        neg = jnp.float32(-jnp.inf)
        bm = jnp.max(jnp.where(segm, e, neg), axis=0, keepdims=True)
        m_old = m_s[...]
        m_new = jnp.maximum(m_old, bm)
        resc = jnp.where(m_old >= m_new, 1.0, jnp.exp(m_old - m_new))
        m_row = jnp.sum(maskf * m_new, axis=1, keepdims=True)
        ex = jnp.exp(e - m_row)
        w = maskf * ex
        s_s[...] = s_s[...] * resc + jnp.sum(w, axis=0, keepdims=True)
        eyeB = (jax.lax.broadcasted_iota(jnp.int32, (_B, _B), 0)
                == jax.lax.broadcasted_iota(jnp.int32, (_B, _B), 1))
        diag_resc = eyeB.astype(jnp.float32) * resc
        acc_s[...] = (jax.lax.dot(diag_resc, acc_s[...])
                      + jax.lax.dot_general(w, fb, _T00))
        m_s[...] = m_new

        @pl.when(i == _NB - 1)
        def _fin():
            sden = s_s[...]
            inv = jnp.where(sden > 0.0, 1.0 / sden, 0.0)
            out_ref[...] = jax.lax.dot(eyeB.astype(jnp.float32) * inv,
                                       acc_s[...])


def kernel(feat, gamma, beta, W_u, W_v, b_v, W_e, segment_ids, last_nodes):
    seg3 = segment_ids.astype(jnp.int32).reshape(_NB, 1, _C)
    ln = last_nodes.astype(jnp.int32).reshape(1, _B)
    g = gamma.reshape(1, _D).astype(jnp.float32)
    bt = beta.reshape(1, _D).astype(jnp.float32)
    bv = b_v.reshape(1, _H).astype(jnp.float32)
    we = W_e.reshape(1, _H).astype(jnp.float32)

    const = lambda p, i: (0, 0)
    out = pl.pallas_call(
        _attn_readout_kernel,
        grid=(2, _NB),
        in_specs=[
            pl.BlockSpec((1, 1, _C), lambda p, i: (i, 0, 0)),
            pl.BlockSpec((1, _B), const),
            pl.BlockSpec((_C, _D), lambda p, i: (i * (1 - p), 0)),
            pl.BlockSpec((1, _D), const),
            pl.BlockSpec((1, _D), const),
            pl.BlockSpec((_D, _H), const),
            pl.BlockSpec((_D, _H), const),
            pl.BlockSpec((1, _H), const),
            pl.BlockSpec((1, _H), const),
        ],
        out_specs=pl.BlockSpec((_B, _D), const),
        out_shape=jax.ShapeDtypeStruct((_B, _D), jnp.float32),
        scratch_shapes=[
            pltpu.VMEM((1, _D), jnp.float32),
            pltpu.VMEM((1, _D), jnp.float32),
            pltpu.VMEM((_B, _D), jnp.float32),
            pltpu.VMEM((1, _B), jnp.float32),
            pltpu.VMEM((1, _B), jnp.float32),
            pltpu.VMEM((_B, _D), jnp.float32),
            pltpu.VMEM((1, _D), jnp.float32),
            pltpu.VMEM((1, _D), jnp.float32),
            pltpu.VMEM((_B, _H), jnp.float32),
            pltpu.VMEM((_N, _D), jnp.float32),
        ],
    )(seg3, ln, feat.astype(jnp.float32), g, bt,
      W_u.astype(jnp.float32), W_v.astype(jnp.float32), bv, we)
    return out
